# Initial kernel scaffold; baseline (speedup 1.0000x reference)
import jax
import jax.numpy as jnp
from jax.experimental import pallas as pl


def kernel(x, edge_index, W, b, a):
    raise NotImplementedError("write your pallas kernel here")


# speedup vs baseline: 5.2961x; 5.2961x over previous
import functools

import jax
import jax.numpy as jnp
from jax import lax
from jax.experimental import pallas as pl
from jax.experimental.pallas import tpu as pltpu
import jax.experimental.pallas.tpu_sc as plsc

N_NODES = 10000
N_EDGES = 320000
DIM = 128
D_EXT = 144
SLOPE = 0.1

N_WORKERS = 32
E_PER_W = N_EDGES // N_WORKERS
BLK = 80
N_BLK = E_PER_W // BLK
ROWS_PER_TILE = N_NODES // 16

ROW_BLOCK = 256
GRID_N = (N_NODES + ROW_BLOCK - 1) // ROW_BLOCK


def _prep_body(x_ref, w_ref, b_ref, a1_ref, a2_ref, ext_ref, s1_ref, s2_ref):
    x = x_ref[...]
    w = w_ref[...]
    emb = lax.dot_general(x, w, (((1,), (1,)), ((), ())),
                          preferred_element_type=jnp.float32)
    emb = emb + b_ref[...]
    s1_ref[...] = jnp.sum(emb * a1_ref[...], axis=1, keepdims=True)
    s2_ref[...] = jnp.sum(emb * a2_ref[...], axis=1, keepdims=True)
    unit = (lax.broadcasted_iota(jnp.int32, (1, D_EXT - DIM), 1) == 0)
    tail = jnp.broadcast_to(unit.astype(jnp.float32),
                            (emb.shape[0], D_EXT - DIM))
    ext_ref[...] = jnp.concatenate([emb, tail], axis=1)


_prep = pl.pallas_call(
    _prep_body,
    grid=(GRID_N,),
    in_specs=[
        pl.BlockSpec((ROW_BLOCK, DIM), lambda i: (i, 0)),
        pl.BlockSpec((DIM, DIM), lambda i: (0, 0)),
        pl.BlockSpec((1, DIM), lambda i: (0, 0)),
        pl.BlockSpec((1, DIM), lambda i: (0, 0)),
        pl.BlockSpec((1, DIM), lambda i: (0, 0)),
    ],
    out_specs=[
        pl.BlockSpec((ROW_BLOCK, D_EXT), lambda i: (i, 0)),
        pl.BlockSpec((ROW_BLOCK, 1), lambda i: (i, 0)),
        pl.BlockSpec((ROW_BLOCK, 1), lambda i: (i, 0)),
    ],
    out_shape=[
        jax.ShapeDtypeStruct((N_NODES, D_EXT), jnp.float32),
        jax.ShapeDtypeStruct((N_NODES, 1), jnp.float32),
        jax.ShapeDtypeStruct((N_NODES, 1), jnp.float32),
    ],
)


_sc_mesh = plsc.VectorSubcoreMesh(core_axis_name="c", subcore_axis_name="s")


@functools.partial(
    pl.kernel,
    out_type=jax.ShapeDtypeStruct((2, N_NODES, D_EXT), jnp.float32),
    mesh=_sc_mesh,
    scratch_types=[
        pltpu.VMEM((BLK,), jnp.int32),
        pltpu.VMEM((BLK,), jnp.int32),
        pltpu.VMEM((N_NODES,), jnp.float32),
        pltpu.VMEM((N_NODES,), jnp.float32),
        pltpu.VMEM((BLK, D_EXT), jnp.float32),
        pltpu.VMEM((BLK,), jnp.float32),
        pltpu.VMEM((25, D_EXT), jnp.float32),
        pltpu.VMEM_SHARED((N_NODES, D_EXT), jnp.float32),
        pltpu.SemaphoreType.DMA,
    ],
    compiler_params=pltpu.CompilerParams(use_tc_tiling_on_sc=False,
                                         needs_layout_passes=False),
)
def _sc_main(ext_hbm, s1_hbm, s2_hbm, src_hbm, dst_hbm, part_hbm,
             src_v, dst_v, s1_v, s2_v, rows_v, vals_v, zero_v, acc, sem):
    c = lax.axis_index("c")
    s = lax.axis_index("s")
    wid = c * 16 + s

    pltpu.sync_copy(s1_hbm, s1_v)
    pltpu.sync_copy(s2_hbm, s2_v)

    def _zrow(r, _):
        for k in range(D_EXT // 16):
            zero_v[r, pl.ds(k * 16, 16)] = jnp.zeros((16,), jnp.float32)
        return _
    lax.fori_loop(0, 25, _zrow, None)

    def _zacc(j, _):
        pltpu.sync_copy(zero_v, acc.at[pl.ds(s * ROWS_PER_TILE + j * 25, 25)])
        return _
    lax.fori_loop(0, ROWS_PER_TILE // 25, _zacc, None)

    plsc.subcore_barrier()

    def _block(bidx, _):
        pltpu.sync_copy(src_hbm.at[wid, bidx], src_v)
        pltpu.sync_copy(dst_hbm.at[wid, bidx], dst_v)
        gather = pltpu.async_copy(ext_hbm.at[dst_v], rows_v, sem)

        for g in range(BLK // 16):
            sv = src_v[pl.ds(g * 16, 16)]
            dv = dst_v[pl.ds(g * 16, 16)]
            logit = (plsc.load_gather(s1_v, [sv]) +
                     plsc.load_gather(s2_v, [dv]))
            vals_v[pl.ds(g * 16, 16)] = jnp.exp(
                jnp.maximum(logit, logit * SLOPE))

        gather.wait()

        def _scale(e, _):
            vv = plsc.load_gather(vals_v, [jnp.full((16,), e, jnp.int32)])
            for k in range(D_EXT // 16):
                sl = pl.ds(k * 16, 16)
                rows_v[e, sl] = rows_v[e, sl] * vv
            return _
        lax.fori_loop(0, BLK, _scale, None)

        pltpu.sync_copy(rows_v, acc.at[src_v], add=True)
        return _

    lax.fori_loop(0, N_BLK, _block, None)

    plsc.subcore_barrier()

    pltpu.sync_copy(acc.at[pl.ds(s * ROWS_PER_TILE, ROWS_PER_TILE)],
                    part_hbm.at[c, pl.ds(s * ROWS_PER_TILE, ROWS_PER_TILE)])


def _combine_body(p0_ref, p1_ref, out_ref):
    tot = p0_ref[...] + p1_ref[...]
    out_ref[...] = tot[:, :DIM] / (tot[:, DIM:DIM + 1] + 1e-12)


_combine = pl.pallas_call(
    _combine_body,
    grid=(GRID_N,),
    in_specs=[
        pl.BlockSpec((ROW_BLOCK, D_EXT), lambda i: (i, 0)),
        pl.BlockSpec((ROW_BLOCK, D_EXT), lambda i: (i, 0)),
    ],
    out_specs=pl.BlockSpec((ROW_BLOCK, DIM), lambda i: (i, 0)),
    out_shape=jax.ShapeDtypeStruct((N_NODES, DIM), jnp.float32),
)


def kernel(x, edge_index, W, b, a):
    src = edge_index[0].astype(jnp.int32).reshape(N_WORKERS, N_BLK, BLK)
    dst = edge_index[1].astype(jnp.int32).reshape(N_WORKERS, N_BLK, BLK)
    a1 = a[:DIM, 0].reshape(1, DIM)
    a2 = a[DIM:, 0].reshape(1, DIM)
    bb = b.reshape(1, DIM)
    ext, s1, s2 = _prep(x, W, bb, a1, a2)
    partial = _sc_main(ext, s1.reshape(-1), s2.reshape(-1), src, dst)
    return _combine(partial[0], partial[1])


# speedup vs baseline: 9.3730x; 1.7698x over previous
import functools

import jax
import jax.numpy as jnp
from jax import lax
from jax.experimental import pallas as pl
from jax.experimental.pallas import tpu as pltpu
import jax.experimental.pallas.tpu_sc as plsc

N_NODES = 10000
N_EDGES = 320000
DIM = 128
D_EXT = 144
SLOPE = 0.1

N_WORKERS = 32
E_PER_W = N_EDGES // N_WORKERS
BLK = 80
N_BLK = E_PER_W // BLK
ROWS_PER_TILE = N_NODES // 16

ROW_BLOCK = 256
GRID_N = (N_NODES + ROW_BLOCK - 1) // ROW_BLOCK


def _prep_body(x_ref, w_ref, b_ref, a1_ref, a2_ref, ext_ref, s1_ref):
    x = x_ref[...]
    w = w_ref[...]
    emb = lax.dot_general(x, w, (((1,), (1,)), ((), ())),
                          preferred_element_type=jnp.float32)
    emb = emb + b_ref[...]
    s1_ref[...] = jnp.sum(emb * a1_ref[...], axis=1, keepdims=True)
    s2 = jnp.sum(emb * a2_ref[...], axis=1, keepdims=True)
    nb = emb.shape[0]
    ones = jnp.ones((nb, 1), jnp.float32)
    zpad = jnp.zeros((nb, D_EXT - DIM - 2), jnp.float32)
    ext_ref[...] = jnp.concatenate([emb, ones, s2, zpad], axis=1)


_prep = pl.pallas_call(
    _prep_body,
    grid=(GRID_N,),
    in_specs=[
        pl.BlockSpec((ROW_BLOCK, DIM), lambda i: (i, 0)),
        pl.BlockSpec((DIM, DIM), lambda i: (0, 0)),
        pl.BlockSpec((1, DIM), lambda i: (0, 0)),
        pl.BlockSpec((1, DIM), lambda i: (0, 0)),
        pl.BlockSpec((1, DIM), lambda i: (0, 0)),
    ],
    out_specs=[
        pl.BlockSpec((ROW_BLOCK, D_EXT), lambda i: (i, 0)),
        pl.BlockSpec((ROW_BLOCK, 1), lambda i: (i, 0)),
    ],
    out_shape=[
        jax.ShapeDtypeStruct((N_NODES, D_EXT), jnp.float32),
        jax.ShapeDtypeStruct((N_NODES, 1), jnp.float32),
    ],
)


_sc_mesh = plsc.VectorSubcoreMesh(core_axis_name="c", subcore_axis_name="s")


@functools.partial(
    pl.kernel,
    out_type=jax.ShapeDtypeStruct((2, N_NODES, D_EXT), jnp.float32),
    mesh=_sc_mesh,
    scratch_types=[
        pltpu.VMEM((4, BLK), jnp.int32),
        pltpu.VMEM((4, BLK), jnp.int32),
        pltpu.VMEM((N_NODES,), jnp.float32),
        pltpu.VMEM((2, BLK, D_EXT), jnp.float32),
        pltpu.VMEM((BLK,), jnp.float32),
        pltpu.VMEM_SHARED((N_NODES, D_EXT), jnp.float32),
        pltpu.SemaphoreType.DMA((4,)),
        pltpu.SemaphoreType.DMA((4,)),
        pltpu.SemaphoreType.DMA((2,)),
        pltpu.SemaphoreType.DMA((2,)),
    ],
    compiler_params=pltpu.CompilerParams(use_tc_tiling_on_sc=False,
                                         needs_layout_passes=False),
)
def _sc_main(ext_hbm, s1_hbm, zeros_hbm, src_hbm, dst_hbm, part_hbm,
             src_v, dst_v, s1_v, rows_v, vals_v, acc,
             sem_si, sem_di, sem_g, sem_sc):
    c = lax.axis_index("c")
    s = lax.axis_index("s")
    wid = c * 16 + s

    pltpu.sync_copy(s1_hbm, s1_v)
    pltpu.sync_copy(zeros_hbm, acc.at[pl.ds(s * ROWS_PER_TILE, ROWS_PER_TILE)])
    plsc.subcore_barrier()

    def _issue_idx(b):
        slot = lax.rem(b, 4)
        pltpu.async_copy(src_hbm.at[wid, b], src_v.at[slot], sem_si.at[slot])
        pltpu.async_copy(dst_hbm.at[wid, b], dst_v.at[slot], sem_di.at[slot])

    def _wait_idx(b):
        slot = lax.rem(b, 4)
        pltpu.make_async_copy(src_hbm.at[wid, b], src_v.at[slot],
                              sem_si.at[slot]).wait()
        pltpu.make_async_copy(dst_hbm.at[wid, b], dst_v.at[slot],
                              sem_di.at[slot]).wait()

    def _start_gather(b, rslot):
        islot = lax.rem(b, 4)
        pltpu.async_copy(ext_hbm.at[dst_v.at[islot]], rows_v.at[rslot],
                         sem_g.at[rslot])

    def _wait_gather(b, rslot):
        islot = lax.rem(b, 4)
        pltpu.make_async_copy(ext_hbm.at[dst_v.at[islot]], rows_v.at[rslot],
                              sem_g.at[rslot]).wait()

    def _start_scatter(b, rslot):
        islot = lax.rem(b, 4)
        pltpu.async_copy(rows_v.at[rslot], acc.at[src_v.at[islot]],
                         sem_sc.at[rslot], add=True)

    def _wait_scatter(b, rslot):
        islot = lax.rem(b, 4)
        pltpu.make_async_copy(rows_v.at[rslot], acc.at[src_v.at[islot]],
                              sem_sc.at[rslot]).wait()

    _issue_idx(0)
    _issue_idx(1)
    _wait_idx(0)
    _start_gather(0, 0)

    def _block(b, _):
        slot = lax.rem(b, 2)
        nslot = 1 - slot

        @pl.when(b + 2 < N_BLK)
        def _():
            _issue_idx(b + 2)

        @pl.when(b + 1 < N_BLK)
        def _():
            _wait_idx(b + 1)

            @pl.when(b >= 1)
            def _():
                _wait_scatter(b - 1, nslot)
            _start_gather(b + 1, nslot)

        _wait_gather(b, slot)

        islot = lax.rem(b, 4)
        lane = lax.iota(jnp.int32, 16)
        for g in range(BLK // 16):
            sv = src_v[islot, pl.ds(g * 16, 16)]
            s2v = plsc.load_gather(
                rows_v.at[slot],
                [lane + g * 16, jnp.full((16,), DIM + 1, jnp.int32)])
            logit = plsc.load_gather(s1_v, [sv]) + s2v
            vals_v[pl.ds(g * 16, 16)] = jnp.exp(
                jnp.maximum(logit, logit * SLOPE))

        def _scale(e, _):
            vv = plsc.load_gather(vals_v, [jnp.full((16,), e, jnp.int32)])
            for k in range(D_EXT // 16):
                sl = pl.ds(k * 16, 16)
                rows_v[slot, e, sl] = rows_v[slot, e, sl] * vv
            return _
        lax.fori_loop(0, BLK, _scale, None)

        _start_scatter(b, slot)
        return _

    lax.fori_loop(0, N_BLK, _block, None)
    _wait_scatter(N_BLK - 1, lax.rem(N_BLK - 1, 2))

    plsc.subcore_barrier()

    pltpu.sync_copy(acc.at[pl.ds(s * ROWS_PER_TILE, ROWS_PER_TILE)],
                    part_hbm.at[c, pl.ds(s * ROWS_PER_TILE, ROWS_PER_TILE)])


def _combine_body(p0_ref, p1_ref, out_ref):
    tot = p0_ref[...] + p1_ref[...]
    out_ref[...] = tot[:, :DIM] / (tot[:, DIM:DIM + 1] + 1e-12)


_combine = pl.pallas_call(
    _combine_body,
    grid=(GRID_N,),
    in_specs=[
        pl.BlockSpec((ROW_BLOCK, D_EXT), lambda i: (i, 0)),
        pl.BlockSpec((ROW_BLOCK, D_EXT), lambda i: (i, 0)),
    ],
    out_specs=pl.BlockSpec((ROW_BLOCK, DIM), lambda i: (i, 0)),
    out_shape=jax.ShapeDtypeStruct((N_NODES, DIM), jnp.float32),
)


def kernel(x, edge_index, W, b, a):
    src = edge_index[0].astype(jnp.int32).reshape(N_WORKERS, N_BLK, BLK)
    dst = edge_index[1].astype(jnp.int32).reshape(N_WORKERS, N_BLK, BLK)
    a1 = a[:DIM, 0].reshape(1, DIM)
    a2 = a[DIM:, 0].reshape(1, DIM)
    bb = b.reshape(1, DIM)
    ext, s1 = _prep(x, W, bb, a1, a2)
    zeros = jnp.zeros((ROWS_PER_TILE, D_EXT), jnp.float32)
    partial = _sc_main(ext, s1.reshape(-1), zeros, src, dst)
    return _combine(partial[0], partial[1])


# speedup vs baseline: 10.7895x; 1.1511x over previous
import functools

import jax
import jax.numpy as jnp
from jax import lax
from jax.experimental import pallas as pl
from jax.experimental.pallas import tpu as pltpu
import jax.experimental.pallas.tpu_sc as plsc

N_NODES = 10000
N_EDGES = 320000
DIM = 128
D_EXT = 144
SLOPE = 0.1

N_WORKERS = 32
E_PER_W = N_EDGES // N_WORKERS
BLK = 80
N_BLK = E_PER_W // BLK
ROWS_PER_TILE = N_NODES // 16

ROW_BLOCK = 1000
GRID_N = (N_NODES + ROW_BLOCK - 1) // ROW_BLOCK


def _prep_body(x_ref, w_ref, b_ref, a1_ref, a2_ref, ext_ref, s1_ref):
    x = x_ref[...]
    w = w_ref[...]
    emb = lax.dot_general(x, w, (((1,), (1,)), ((), ())),
                          preferred_element_type=jnp.float32)
    emb = emb + b_ref[...]
    s1_ref[...] = jnp.sum(emb * a1_ref[...], axis=1, keepdims=True)
    s2 = jnp.sum(emb * a2_ref[...], axis=1, keepdims=True)
    nb = emb.shape[0]
    ones = jnp.ones((nb, 1), jnp.float32)
    zpad = jnp.zeros((nb, D_EXT - DIM - 2), jnp.float32)
    ext_ref[...] = jnp.concatenate([emb, ones, s2, zpad], axis=1)


_prep = pl.pallas_call(
    _prep_body,
    grid=(GRID_N,),
    in_specs=[
        pl.BlockSpec((ROW_BLOCK, DIM), lambda i: (i, 0)),
        pl.BlockSpec((DIM, DIM), lambda i: (0, 0)),
        pl.BlockSpec((1, DIM), lambda i: (0, 0)),
        pl.BlockSpec((1, DIM), lambda i: (0, 0)),
        pl.BlockSpec((1, DIM), lambda i: (0, 0)),
    ],
    out_specs=[
        pl.BlockSpec((ROW_BLOCK, D_EXT), lambda i: (i, 0)),
        pl.BlockSpec((ROW_BLOCK, 1), lambda i: (i, 0)),
    ],
    out_shape=[
        jax.ShapeDtypeStruct((N_NODES, D_EXT), jnp.float32),
        jax.ShapeDtypeStruct((N_NODES, 1), jnp.float32),
    ],
)


_sc_mesh = plsc.VectorSubcoreMesh(core_axis_name="c", subcore_axis_name="s")


@functools.partial(
    pl.kernel,
    out_type=jax.ShapeDtypeStruct((2, N_NODES, D_EXT), jnp.float32),
    mesh=_sc_mesh,
    scratch_types=[
        pltpu.VMEM((4, BLK), jnp.int32),
        pltpu.VMEM((4, BLK), jnp.int32),
        pltpu.VMEM((N_NODES,), jnp.float32),
        pltpu.VMEM((2, BLK, D_EXT), jnp.float32),
        pltpu.VMEM((BLK,), jnp.float32),
        pltpu.VMEM_SHARED((N_NODES, D_EXT), jnp.float32),
        pltpu.SemaphoreType.DMA((4,)),
        pltpu.SemaphoreType.DMA((4,)),
        pltpu.SemaphoreType.DMA((2,)),
        pltpu.SemaphoreType.DMA((2,)),
    ],
    compiler_params=pltpu.CompilerParams(use_tc_tiling_on_sc=False,
                                         needs_layout_passes=False),
)
def _sc_main(ext_hbm, s1_hbm, zeros_hbm, edges_hbm, part_hbm,
             src_v, dst_v, s1_v, rows_v, vals_v, acc,
             sem_si, sem_di, sem_g, sem_sc):
    c = lax.axis_index("c")
    s = lax.axis_index("s")
    wid = c * 16 + s

    pltpu.sync_copy(s1_hbm, s1_v)
    pltpu.sync_copy(zeros_hbm, acc.at[pl.ds(s * ROWS_PER_TILE, ROWS_PER_TILE)])
    plsc.subcore_barrier()

    def _issue_idx(b):
        slot = lax.rem(b, 4)
        pltpu.async_copy(edges_hbm.at[0, wid, b], src_v.at[slot],
                         sem_si.at[slot])
        pltpu.async_copy(edges_hbm.at[1, wid, b], dst_v.at[slot],
                         sem_di.at[slot])

    def _wait_idx(b):
        slot = lax.rem(b, 4)
        pltpu.make_async_copy(edges_hbm.at[0, wid, b], src_v.at[slot],
                              sem_si.at[slot]).wait()
        pltpu.make_async_copy(edges_hbm.at[1, wid, b], dst_v.at[slot],
                              sem_di.at[slot]).wait()

    def _start_gather(b, rslot):
        islot = lax.rem(b, 4)
        pltpu.async_copy(ext_hbm.at[dst_v.at[islot]], rows_v.at[rslot],
                         sem_g.at[rslot])

    def _wait_gather(b, rslot):
        islot = lax.rem(b, 4)
        pltpu.make_async_copy(ext_hbm.at[dst_v.at[islot]], rows_v.at[rslot],
                              sem_g.at[rslot]).wait()

    def _start_scatter(b, rslot):
        islot = lax.rem(b, 4)
        pltpu.async_copy(rows_v.at[rslot], acc.at[src_v.at[islot]],
                         sem_sc.at[rslot], add=True)

    def _wait_scatter(b, rslot):
        islot = lax.rem(b, 4)
        pltpu.make_async_copy(rows_v.at[rslot], acc.at[src_v.at[islot]],
                              sem_sc.at[rslot]).wait()

    _issue_idx(0)
    _issue_idx(1)
    _wait_idx(0)
    _start_gather(0, 0)

    def _block(b, _):
        slot = lax.rem(b, 2)
        nslot = 1 - slot

        @pl.when(b + 2 < N_BLK)
        def _():
            _issue_idx(b + 2)

        @pl.when(b + 1 < N_BLK)
        def _():
            _wait_idx(b + 1)

            @pl.when(b >= 1)
            def _():
                _wait_scatter(b - 1, nslot)
            _start_gather(b + 1, nslot)

        _wait_gather(b, slot)

        islot = lax.rem(b, 4)
        lane = lax.iota(jnp.int32, 16)
        for g in range(BLK // 16):
            sv = src_v[islot, pl.ds(g * 16, 16)]
            s2v = plsc.load_gather(
                rows_v.at[slot],
                [lane + g * 16, jnp.full((16,), DIM + 1, jnp.int32)])
            logit = plsc.load_gather(s1_v, [sv]) + s2v
            vals_v[pl.ds(g * 16, 16)] = jnp.exp(
                jnp.maximum(logit, logit * SLOPE))

        def _scale(e4, _):
            for u in range(4):
                e = e4 * 4 + u
                vv = plsc.load_gather(vals_v, [jnp.full((16,), e, jnp.int32)])
                for k in range(D_EXT // 16):
                    sl = pl.ds(k * 16, 16)
                    rows_v[slot, e, sl] = rows_v[slot, e, sl] * vv
            return _
        lax.fori_loop(0, BLK // 4, _scale, None, unroll=1)

        _start_scatter(b, slot)
        return _

    lax.fori_loop(0, N_BLK, _block, None)
    _wait_scatter(N_BLK - 1, lax.rem(N_BLK - 1, 2))

    plsc.subcore_barrier()

    pltpu.sync_copy(acc.at[pl.ds(s * ROWS_PER_TILE, ROWS_PER_TILE)],
                    part_hbm.at[c, pl.ds(s * ROWS_PER_TILE, ROWS_PER_TILE)])


def _combine_body(p0_ref, p1_ref, out_ref):
    tot = p0_ref[...] + p1_ref[...]
    out_ref[...] = tot[:, :DIM] / (tot[:, DIM:DIM + 1] + 1e-12)


_combine = pl.pallas_call(
    _combine_body,
    grid=(GRID_N,),
    in_specs=[
        pl.BlockSpec((ROW_BLOCK, D_EXT), lambda i: (i, 0)),
        pl.BlockSpec((ROW_BLOCK, D_EXT), lambda i: (i, 0)),
    ],
    out_specs=pl.BlockSpec((ROW_BLOCK, DIM), lambda i: (i, 0)),
    out_shape=jax.ShapeDtypeStruct((N_NODES, DIM), jnp.float32),
)


def kernel(x, edge_index, W, b, a):
    edges = edge_index.astype(jnp.int32).reshape(2, N_WORKERS, N_BLK, BLK)
    a1 = a[:DIM, 0].reshape(1, DIM)
    a2 = a[DIM:, 0].reshape(1, DIM)
    bb = b.reshape(1, DIM)
    ext, s1 = _prep(x, W, bb, a1, a2)
    zeros = jnp.zeros((ROWS_PER_TILE, D_EXT), jnp.float32)
    partial = _sc_main(ext, s1.reshape(-1), zeros, edges)
    return _combine(partial[0], partial[1])
